# baseline (speedup 1.0000x reference)
import functools

import jax
import jax.numpy as jnp
from jax import lax
from jax.experimental import pallas as pl
from jax.experimental.pallas import tpu as pltpu
from jax.experimental.pallas import tpu_sc as plsc

NC = 2
NS = 16
NW = NC * NS
LANES = 16

C_SEQ = 2
SUB = 100


def _positional_encoding(seq_len: int, d_model: int) -> jax.Array:
    pos = jnp.arange(seq_len, dtype=jnp.float32)[:, None]
    _2i = jnp.arange(0, d_model, 2, dtype=jnp.float32)
    enc = jnp.zeros((seq_len, d_model), dtype=jnp.float32)
    enc = enc.at[:, 0::2].set(jnp.sin(pos / (10000.0 ** (_2i / d_model))))
    enc = enc.at[:, 1::2].set(jnp.cos(pos / (10000.0 ** (_2i / d_model))))
    return enc


@functools.partial(jax.jit, static_argnames=("B", "S", "D"))
def _embed_sc(idx2d, tbl128, enc2, *, B, S, D):
    N = B * S
    R = C_SEQ * S
    KSUB = R // SUB
    rows_per_w = N // NW
    G = rows_per_w // R
    srows_per_w = rows_per_w // SUB

    mesh = plsc.VectorSubcoreMesh(core_axis_name="c", subcore_axis_name="s")

    @functools.partial(
        pl.kernel,
        mesh=mesh,
        out_type=jax.ShapeDtypeStruct((N, D), jnp.float32),
        scratch_types=[
            pltpu.VMEM((KSUB, SUB), jnp.int32),
            pltpu.VMEM((R, 128), jnp.float32),
            pltpu.VMEM((R, D), jnp.float32),
            pltpu.VMEM((S // 2, 128), jnp.float32),
            pltpu.SemaphoreType.DMA,
        ],
    )
    def body(idx_hbm, table_hbm, enc_hbm, out_hbm, idx_v, gbuf_v, obuf_v,
             enc_v, sem):
        wid = lax.axis_index("s") * NC + lax.axis_index("c")
        pltpu.sync_copy(enc_hbm, enc_v)

        def chunk(g, carry):
            row0 = wid * rows_per_w + g * R
            srow0 = wid * srows_per_w + g * KSUB
            pltpu.sync_copy(idx_hbm.at[pl.ds(srow0, KSUB), :], idx_v)
            cps = [
                pltpu.async_copy(
                    table_hbm.at[idx_v.at[k]],
                    gbuf_v.at[pl.ds(k * SUB, SUB), :],
                    sem,
                )
                for k in range(KSUB)
            ]
            for cp in cps:
                cp.wait()

            def add_row(s2, c2):
                for col in range(128 // LANES):
                    dsl = pl.ds((col % (D // LANES)) * LANES, LANES)
                    e = enc_v[s2, pl.ds(col * LANES, LANES)]
                    for c in range(C_SEQ):
                        r = c * S + 2 * s2 + col // (D // LANES)
                        obuf_v[r, dsl] = gbuf_v[r, dsl] + e
                return c2

            lax.fori_loop(0, S // 2, add_row, 0)
            pltpu.sync_copy(obuf_v, out_hbm.at[pl.ds(row0, R), :])
            return carry

        lax.fori_loop(0, G, chunk, 0)

    return body(idx2d, tbl128, enc2)


def kernel(x, table):
    B, S = x.shape
    _, D = table.shape
    N = B * S
    idx2d = x.reshape(N // SUB, SUB)
    tbl128 = jnp.pad(table, ((0, 0), (0, 128 - D)))
    enc2 = _positional_encoding(S, D).reshape(S // 2, 2 * D)
    out = _embed_sc(idx2d, tbl128, enc2, B=B, S=S, D=D)
    return out.reshape(B, S, D)


# speedup vs baseline: 1.0024x; 1.0024x over previous
import functools

import jax
import jax.numpy as jnp
from jax import lax
from jax.experimental import pallas as pl
from jax.experimental.pallas import tpu as pltpu
from jax.experimental.pallas import tpu_sc as plsc

NC = 2
NS = 16
NW = NC * NS
LANES = 16

C_SEQ = 2
SUB = 100


def _positional_encoding(seq_len: int, d_model: int) -> jax.Array:
    pos = jnp.arange(seq_len, dtype=jnp.float32)[:, None]
    _2i = jnp.arange(0, d_model, 2, dtype=jnp.float32)
    enc = jnp.zeros((seq_len, d_model), dtype=jnp.float32)
    enc = enc.at[:, 0::2].set(jnp.sin(pos / (10000.0 ** (_2i / d_model))))
    enc = enc.at[:, 1::2].set(jnp.cos(pos / (10000.0 ** (_2i / d_model))))
    return enc


@functools.partial(jax.jit, static_argnames=("B", "S", "D"))
def _embed_sc(idx2d, table, enc, *, B, S, D):
    N = B * S
    R = C_SEQ * S
    KSUB = R // SUB
    rows_per_w = N // NW
    seqs_per_w = rows_per_w // S
    G = rows_per_w // R
    srows_per_w = rows_per_w // SUB

    mesh = plsc.VectorSubcoreMesh(core_axis_name="c", subcore_axis_name="s")

    @functools.partial(
        pl.kernel,
        mesh=mesh,
        compiler_params=pltpu.CompilerParams(use_tc_tiling_on_sc=False),
        out_type=jax.ShapeDtypeStruct((B, S, D), jnp.float32),
        scratch_types=[
            pltpu.VMEM((KSUB, SUB), jnp.int32),
            pltpu.VMEM((R, D), jnp.float32),
            pltpu.VMEM((C_SEQ, S, D), jnp.float32),
            pltpu.VMEM((S, D), jnp.float32),
            pltpu.SemaphoreType.DMA,
        ],
    )
    def body(idx_hbm, table_hbm, enc_hbm, out_hbm, idx_v, gbuf_v, obuf_v,
             enc_v, sem):
        wid = lax.axis_index("s") * NC + lax.axis_index("c")
        pltpu.sync_copy(enc_hbm, enc_v)

        def chunk(g, carry):
            b0 = wid * seqs_per_w + g * C_SEQ
            srow0 = wid * srows_per_w + g * KSUB
            pltpu.sync_copy(idx_hbm.at[pl.ds(srow0, KSUB), :], idx_v)
            cps = [
                pltpu.async_copy(
                    table_hbm.at[idx_v.at[k]],
                    gbuf_v.at[pl.ds(k * SUB, SUB), :],
                    sem,
                )
                for k in range(KSUB)
            ]
            for cp in cps:
                cp.wait()

            def add_row(s, c2):
                for d in range(D // LANES):
                    sl = pl.ds(d * LANES, LANES)
                    e = enc_v[s, sl]
                    for c in range(C_SEQ):
                        obuf_v[c, s, sl] = gbuf_v[c * S + s, sl] + e
                return c2

            lax.fori_loop(0, S, add_row, 0)
            pltpu.sync_copy(obuf_v, out_hbm.at[pl.ds(b0, C_SEQ)])
            return carry

        lax.fori_loop(0, G, chunk, 0)

    return body(idx2d, table, enc)


def kernel(x, table):
    B, S = x.shape
    _, D = table.shape
    idx2d = x.reshape(B * S // SUB, SUB)
    enc = _positional_encoding(S, D)
    return _embed_sc(idx2d, table, enc, B=B, S=S, D=D)


# speedup vs baseline: 1.6893x; 1.6852x over previous
import functools

import jax
import jax.numpy as jnp
from jax import lax
from jax.experimental import pallas as pl
from jax.experimental.pallas import tpu as pltpu
from jax.experimental.pallas import tpu_sc as plsc
from jax.experimental.layout import Format, Layout, with_layout_constraint

NC = 2
NS = 16
NW = NC * NS
LANES = 16

C_SEQ = 2
SUB = 100


def _positional_encoding(seq_len: int, d_model: int) -> jax.Array:
    pos = jnp.arange(seq_len, dtype=jnp.float32)[:, None]
    _2i = jnp.arange(0, d_model, 2, dtype=jnp.float32)
    enc = jnp.zeros((seq_len, d_model), dtype=jnp.float32)
    enc = enc.at[:, 0::2].set(jnp.sin(pos / (10000.0 ** (_2i / d_model))))
    enc = enc.at[:, 1::2].set(jnp.cos(pos / (10000.0 ** (_2i / d_model))))
    return enc


@functools.partial(jax.jit, static_argnames=("B", "S", "D"))
def _embed_sc(idx2d, tbl128, enc, *, B, S, D):
    N = B * S
    R = C_SEQ * S
    KSUB = R // SUB
    rows_per_w = N // NW
    seqs_per_w = rows_per_w // S
    G = rows_per_w // R
    srows_per_w = rows_per_w // SUB

    mesh = plsc.VectorSubcoreMesh(core_axis_name="c", subcore_axis_name="s")

    @functools.partial(
        pl.kernel,
        mesh=mesh,
        out_type=jax.ShapeDtypeStruct((B, S, D), jnp.float32),
        scratch_types=[
            pltpu.VMEM((KSUB, SUB), jnp.int32),
            pltpu.VMEM((R, 128), jnp.float32),
            pltpu.VMEM((C_SEQ, S, D), jnp.float32),
            pltpu.VMEM((S, D), jnp.float32),
            pltpu.SemaphoreType.DMA,
        ],
    )
    def body(idx_hbm, table_hbm, enc_hbm, out_hbm, idx_v, gbuf_v, obuf_v,
             enc_v, sem):
        wid = lax.axis_index("s") * NC + lax.axis_index("c")
        pltpu.sync_copy(enc_hbm, enc_v)

        def chunk(g, carry):
            b0 = wid * seqs_per_w + g * C_SEQ
            srow0 = wid * srows_per_w + g * KSUB
            pltpu.sync_copy(idx_hbm.at[pl.ds(srow0, KSUB), :], idx_v)
            cps = [
                pltpu.async_copy(
                    table_hbm.at[idx_v.at[k]],
                    gbuf_v.at[pl.ds(k * SUB, SUB), :],
                    sem,
                )
                for k in range(KSUB)
            ]
            for cp in cps:
                cp.wait()

            def add_row(s, c2):
                for d in range(D // LANES):
                    sl = pl.ds(d * LANES, LANES)
                    e = enc_v[s, sl]
                    for c in range(C_SEQ):
                        obuf_v[c, s, sl] = gbuf_v[c * S + s, sl] + e
                return c2

            lax.fori_loop(0, S, add_row, 0)
            pltpu.sync_copy(obuf_v, out_hbm.at[pl.ds(b0, C_SEQ)])
            return carry

        lax.fori_loop(0, G, chunk, 0)

    return body(idx2d, tbl128, enc)


def kernel(x, table):
    B, S = x.shape
    _, D = table.shape
    idx2d = x.reshape(B * S // SUB, SUB)
    tbl128 = jnp.pad(table, ((0, 0), (0, 128 - D)))
    enc = _positional_encoding(S, D)
    out = _embed_sc(idx2d, tbl128, enc, B=B, S=S, D=D)
    return with_layout_constraint(out, Layout((0, 1, 2)))
